# baseline (speedup 1.0000x reference)
import functools

import jax
import jax.numpy as jnp
from jax import lax
from jax.experimental import pallas as pl
from jax.experimental.pallas import tpu as pltpu
from jax.experimental.pallas import tpu_sc as plsc

NC, NS, L = 2, 16, 16
NW = NC * NS
D = 64
C = 128
NBUF = 5
K = 3
TB = 5


def _tile_body(idx_hbm, table_hbm, out_hbm, idx_v, rows_v, tp_v, flag_v, *sems):
    gsem = sems[:NBUF]
    tsem = sems[NBUF:]
    wid = lax.axis_index("s") * NC + lax.axis_index("c")
    n_idx = idx_hbm.shape[0]
    n_rows = n_idx // NW
    n_chunks = n_rows // C
    base = wid * n_rows

    pltpu.sync_copy(idx_hbm.at[pl.ds(base, n_rows)], idx_v.at[pl.ds(0, n_rows)])

    def gather_copy(j, b):
        return pltpu.make_async_copy(
            table_hbm.at[idx_v.at[pl.ds(j * C, C)]], rows_v.at[b], gsem[b]
        )

    def out_copy(j, tb):
        q = wid * n_chunks + j
        nbg = out_hbm.shape[1] // C
        s = q // nbg
        bg = q % nbg
        return pltpu.make_async_copy(
            tp_v.at[tb, :, pl.ds(0, C)],
            out_hbm.at[pl.ds(s * D, D), pl.ds(bg * C, C)],
            tsem[tb],
        )

    def fixup(j, b):
        acc = jnp.zeros((L,), jnp.bool_)
        for g in range(C // L):
            ig = idx_v[pl.ds(j * C + g * L, L)]
            acc = acc | (ig == 0)
        flag_v[...] = jnp.zeros((L,), jnp.int32)
        plsc.store_scatter(
            flag_v, [jnp.zeros((L,), jnp.int32)], jnp.ones((L,), jnp.int32), mask=acc
        )
        haspad = flag_v[...][0]

        @pl.when(haspad > 0)
        def _fix():
            @pl.loop(0, C)
            def _row(r):
                v = idx_v[pl.ds(j * C + r, L)][0]

                @pl.when(v == 0)
                def _zero_row():
                    z = jnp.zeros((L,), jnp.float32)
                    rr = jnp.full((L,), r, jnp.int32)
                    col = lax.iota(jnp.int32, L)
                    for cg in range(D // L):
                        plsc.store_scatter(rows_v.at[b], [rr, col + cg * L], z)

    row_ids = [lax.iota(jnp.int32, L) + g * L for g in range(C // L)]

    def transpose(b, tb):
        @pl.loop(0, C, unroll=8)
        def _r(r):
            colf = jnp.full((L,), r, jnp.int32)
            for g in range(D // L):
                vr = rows_v[b, r, pl.ds(g * L, L)]
                plsc.store_scatter(tp_v.at[tb], [row_ids[g], colf], vr)

    def step(j, b, tb):
        gather_copy(j, b).wait()
        fixup(j, b)

        @pl.when(j >= TB)
        def _drain():
            out_copy(0, tb).wait()

        transpose(b, tb)
        out_copy(j, tb).start()

        @pl.when(j + K < n_chunks)
        def _fire():
            gather_copy(j + K, (b + K) % NBUF).start()

    for jf in range(K):
        gather_copy(jf, jf % NBUF).start()

    @pl.loop(0, n_chunks // NBUF)
    def _grp(o):
        for u in range(NBUF):
            step(o * NBUF + u, u, u % TB)

    for u in range(TB):
        out_copy(0, u).wait()


def kernel(url_ids, table):
    batch, seq = url_ids.shape
    idx = jnp.swapaxes(url_ids, 0, 1).reshape(-1).astype(jnp.int32)
    n = idx.shape[0]
    n_rows = n // NW

    mesh = plsc.VectorSubcoreMesh(
        core_axis_name="c", subcore_axis_name="s", num_cores=NC, num_subcores=NS
    )
    f = pl.kernel(
        _tile_body,
        out_type=jax.ShapeDtypeStruct((seq * D, batch), jnp.float32),
        mesh=mesh,
        compiler_params=pltpu.CompilerParams(
            needs_layout_passes=False, use_tc_tiling_on_sc=False
        ),
        scratch_types=[
            pltpu.VMEM((n_rows + L,), jnp.int32),
            pltpu.VMEM((NBUF, C, D), jnp.float32),
            pltpu.VMEM((TB, D, C + 1), jnp.float32),
            pltpu.VMEM((L,), jnp.int32),
        ]
        + [pltpu.SemaphoreType.DMA] * (NBUF + TB),
    )
    out = f(idx, table)
    return out.reshape(seq, D, batch).transpose(2, 0, 1)


# speedup vs baseline: 1.0164x; 1.0164x over previous
import functools

import jax
import jax.numpy as jnp
from jax import lax
from jax.experimental import pallas as pl
from jax.experimental.pallas import tpu as pltpu
from jax.experimental.pallas import tpu_sc as plsc

NC, NS, L = 2, 16, 16
NW = NC * NS
D = 64
C = 128
NBUF = 2
K = 2
TB = 2


def _tile_body(idx_hbm, table_hbm, out_hbm, idx_v, rows_v, tp_v, flag_v, *sems):
    gsem = sems[:NBUF]
    tsem = sems[NBUF:]
    wid = lax.axis_index("s") * NC + lax.axis_index("c")
    n_idx = idx_hbm.shape[0]
    n_rows = n_idx // NW
    n_chunks = n_rows // C
    base = wid * n_rows

    pltpu.sync_copy(idx_hbm.at[pl.ds(base, n_rows)], idx_v.at[pl.ds(0, n_rows)])

    def gather_copy(j, b):
        return pltpu.make_async_copy(
            table_hbm.at[idx_v.at[pl.ds(j * C, C)]], rows_v.at[b], gsem[b]
        )

    def out_copy(j, tb):
        q = wid * n_chunks + j
        nbg = out_hbm.shape[1] // C
        s = q // nbg
        bg = q % nbg
        return pltpu.make_async_copy(
            tp_v.at[tb, :, pl.ds(0, C)],
            out_hbm.at[pl.ds(s * D, D), pl.ds(bg * C, C)],
            tsem[tb],
        )

    def fixup(j, b):
        acc = jnp.zeros((L,), jnp.bool_)
        for g in range(C // L):
            ig = idx_v[pl.ds(j * C + g * L, L)]
            acc = acc | (ig == 0)
        flag_v[...] = jnp.zeros((L,), jnp.int32)
        plsc.store_scatter(
            flag_v, [jnp.zeros((L,), jnp.int32)], jnp.ones((L,), jnp.int32), mask=acc
        )
        haspad = flag_v[...][0]

        @pl.when(haspad > 0)
        def _fix():
            @pl.loop(0, C)
            def _row(r):
                v = idx_v[pl.ds(j * C + r, L)][0]

                @pl.when(v == 0)
                def _zero_row():
                    z = jnp.zeros((L,), jnp.float32)
                    rr = jnp.full((L,), r, jnp.int32)
                    col = lax.iota(jnp.int32, L)
                    for cg in range(D // L):
                        plsc.store_scatter(rows_v.at[b], [rr, col + cg * L], z)

    row_ids = [lax.iota(jnp.int32, L) + g * L for g in range(C // L)]

    def transpose(b, tb):
        @pl.loop(0, C, unroll=8)
        def _r(r):
            colf = jnp.full((L,), r, jnp.int32)
            for g in range(D // L):
                vr = rows_v[b, r, pl.ds(g * L, L)]
                plsc.store_scatter(tp_v.at[tb], [row_ids[g], colf], vr)

    def step(j, b, tb):
        gather_copy(j, b).wait()
        fixup(j, b)

        @pl.when(j >= TB)
        def _drain():
            out_copy(0, tb).wait()

        transpose(b, tb)
        out_copy(j, tb).start()

        @pl.when(j + K < n_chunks)
        def _fire():
            gather_copy(j + K, (b + K) % NBUF).start()

    for jf in range(K):
        gather_copy(jf, jf % NBUF).start()

    @pl.loop(0, n_chunks // NBUF)
    def _grp(o):
        for u in range(NBUF):
            step(o * NBUF + u, u, u % TB)

    for u in range(TB):
        out_copy(0, u).wait()


def kernel(url_ids, table):
    batch, seq = url_ids.shape
    idx = jnp.swapaxes(url_ids, 0, 1).reshape(-1).astype(jnp.int32)
    n = idx.shape[0]
    n_rows = n // NW

    mesh = plsc.VectorSubcoreMesh(
        core_axis_name="c", subcore_axis_name="s", num_cores=NC, num_subcores=NS
    )
    f = pl.kernel(
        _tile_body,
        out_type=jax.ShapeDtypeStruct((seq * D, batch), jnp.float32),
        mesh=mesh,
        compiler_params=pltpu.CompilerParams(
            needs_layout_passes=False, use_tc_tiling_on_sc=False
        ),
        scratch_types=[
            pltpu.VMEM((n_rows + L,), jnp.int32),
            pltpu.VMEM((NBUF, C, D), jnp.float32),
            pltpu.VMEM((TB, D, C + 1), jnp.float32),
            pltpu.VMEM((L,), jnp.int32),
        ]
        + [pltpu.SemaphoreType.DMA] * (NBUF + TB),
    )
    out = f(idx, table)
    return out.reshape(seq, D, batch).transpose(2, 0, 1)


# speedup vs baseline: 1.3423x; 1.3207x over previous
import functools

import jax
import jax.numpy as jnp
from jax import lax
from jax.experimental import pallas as pl
from jax.experimental.pallas import tpu as pltpu
from jax.experimental.pallas import tpu_sc as plsc

NC, NS, L = 2, 16, 16
NW = NC * NS
D = 64
C = 128
NBUF = 2
K = 2
TB = 2


def _tile_body(idx_hbm, table_hbm, out_hbm, idx_v, rows_v, tp_v, flag_v, *sems):
    gsem = sems[:NBUF]
    tsem = sems[NBUF:]
    wid = lax.axis_index("s") * NC + lax.axis_index("c")
    n_idx = idx_hbm.shape[0]
    n_rows = n_idx // NW
    n_chunks = n_rows // C
    base = wid * n_rows

    pltpu.sync_copy(idx_hbm.at[pl.ds(base, n_rows)], idx_v.at[pl.ds(0, n_rows)])

    def gather_copy(j, b):
        return pltpu.make_async_copy(
            table_hbm.at[idx_v.at[pl.ds(j * C, C)]], rows_v.at[b], gsem[b]
        )

    def out_copy(j, tb):
        q = wid * n_chunks + j
        nbg = out_hbm.shape[1] // C
        s = q // nbg
        bg = q % nbg
        return pltpu.make_async_copy(
            tp_v.at[tb, :, pl.ds(0, C)],
            out_hbm.at[pl.ds(s * D, D), pl.ds(bg * C, C)],
            tsem[tb],
        )

    def fixup(j, b):
        acc = jnp.zeros((L,), jnp.bool_)
        for g in range(C // L):
            ig = idx_v[pl.ds(j * C + g * L, L)]
            acc = acc | (ig == 0)
        flag_v[...] = jnp.zeros((L,), jnp.int32)
        plsc.store_scatter(
            flag_v, [jnp.zeros((L,), jnp.int32)], jnp.ones((L,), jnp.int32), mask=acc
        )
        haspad = flag_v[...][0]

        @pl.when(haspad > 0)
        def _fix():
            @pl.loop(0, C)
            def _row(r):
                v = idx_v[pl.ds(j * C + r, L)][0]

                @pl.when(v == 0)
                def _zero_row():
                    z = jnp.zeros((L,), jnp.float32)
                    rr = jnp.full((L,), r, jnp.int32)
                    col = lax.iota(jnp.int32, L)
                    for cg in range(D // L):
                        plsc.store_scatter(rows_v.at[b], [rr, col + cg * L], z)

    row_ids = [lax.iota(jnp.int32, L) + g * L for g in range(C // L)]

    def transpose(b, tb):
        @plsc.parallel_loop(0, C, unroll=8)
        def _r(r):
            colf = jnp.full((L,), r, jnp.int32)
            for g in range(D // L):
                vr = rows_v[b, r, pl.ds(g * L, L)]
                plsc.store_scatter(tp_v.at[tb], [row_ids[g], colf], vr)

    def step(j, b, tb):
        gather_copy(j, b).wait()
        fixup(j, b)

        @pl.when(j >= TB)
        def _drain():
            out_copy(0, tb).wait()

        transpose(b, tb)
        out_copy(j, tb).start()

        @pl.when(j + K < n_chunks)
        def _fire():
            gather_copy(j + K, (b + K) % NBUF).start()

    for jf in range(K):
        gather_copy(jf, jf % NBUF).start()

    @pl.loop(0, n_chunks // NBUF)
    def _grp(o):
        for u in range(NBUF):
            step(o * NBUF + u, u, u % TB)

    for u in range(TB):
        out_copy(0, u).wait()


def kernel(url_ids, table):
    batch, seq = url_ids.shape
    idx = jnp.swapaxes(url_ids, 0, 1).reshape(-1).astype(jnp.int32)
    n = idx.shape[0]
    n_rows = n // NW

    mesh = plsc.VectorSubcoreMesh(
        core_axis_name="c", subcore_axis_name="s", num_cores=NC, num_subcores=NS
    )
    f = pl.kernel(
        _tile_body,
        out_type=jax.ShapeDtypeStruct((seq * D, batch), jnp.float32),
        mesh=mesh,
        compiler_params=pltpu.CompilerParams(
            needs_layout_passes=False, use_tc_tiling_on_sc=False
        ),
        scratch_types=[
            pltpu.VMEM((n_rows + L,), jnp.int32),
            pltpu.VMEM((NBUF, C, D), jnp.float32),
            pltpu.VMEM((TB, D, C + 1), jnp.float32),
            pltpu.VMEM((L,), jnp.int32),
        ]
        + [pltpu.SemaphoreType.DMA] * (NBUF + TB),
    )
    out = f(idx, table)
    return out.reshape(seq, D, batch).transpose(2, 0, 1)


# speedup vs baseline: 1.4108x; 1.0510x over previous
import functools

import jax
import jax.numpy as jnp
from jax import lax
from jax.experimental import pallas as pl
from jax.experimental.pallas import tpu as pltpu
from jax.experimental.pallas import tpu_sc as plsc

NC, NS, L = 2, 16, 16
NW = NC * NS
D = 64
C = 128
NBUF = 5
K = 3
TB = 5


def _tile_body(idx_hbm, table_hbm, out_hbm, idx_v, rows_v, tp_v, flag_v, *sems):
    gsem = sems[:NBUF]
    tsem = sems[NBUF:]
    wid = lax.axis_index("s") * NC + lax.axis_index("c")
    n_idx = idx_hbm.shape[0]
    n_rows = n_idx // NW
    n_chunks = n_rows // C
    base = wid * n_rows

    pltpu.sync_copy(idx_hbm.at[pl.ds(base, n_rows)], idx_v.at[pl.ds(0, n_rows)])

    def gather_copy(j, b):
        return pltpu.make_async_copy(
            table_hbm.at[idx_v.at[pl.ds(j * C, C)]], rows_v.at[b], gsem[b]
        )

    def out_copy(j, tb):
        q = wid * n_chunks + j
        nbg = out_hbm.shape[1] // C
        s = q // nbg
        bg = q % nbg
        return pltpu.make_async_copy(
            tp_v.at[tb, :, pl.ds(0, C)],
            out_hbm.at[pl.ds(s * D, D), pl.ds(bg * C, C)],
            tsem[tb],
        )

    def fixup(j, b):
        acc = jnp.zeros((L,), jnp.bool_)
        for g in range(C // L):
            ig = idx_v[pl.ds(j * C + g * L, L)]
            acc = acc | (ig == 0)
        flag_v[...] = jnp.zeros((L,), jnp.int32)
        plsc.store_scatter(
            flag_v, [jnp.zeros((L,), jnp.int32)], jnp.ones((L,), jnp.int32), mask=acc
        )
        haspad = flag_v[...][0]

        @pl.when(haspad > 0)
        def _fix():
            @pl.loop(0, C)
            def _row(r):
                v = idx_v[pl.ds(j * C + r, L)][0]

                @pl.when(v == 0)
                def _zero_row():
                    z = jnp.zeros((L,), jnp.float32)
                    rr = jnp.full((L,), r, jnp.int32)
                    col = lax.iota(jnp.int32, L)
                    for cg in range(D // L):
                        plsc.store_scatter(rows_v.at[b], [rr, col + cg * L], z)

    row_ids = [lax.iota(jnp.int32, L) + g * L for g in range(C // L)]

    def transpose(b, tb):
        @plsc.parallel_loop(0, C, unroll=8)
        def _r(r):
            colf = jnp.full((L,), r, jnp.int32)
            for g in range(D // L):
                vr = rows_v[b, r, pl.ds(g * L, L)]
                plsc.store_scatter(tp_v.at[tb], [row_ids[g], colf], vr)

    def step(j, b, tb):
        gather_copy(j, b).wait()
        fixup(j, b)

        @pl.when(j >= TB)
        def _drain():
            out_copy(0, tb).wait()

        transpose(b, tb)
        out_copy(j, tb).start()

        @pl.when(j + K < n_chunks)
        def _fire():
            gather_copy(j + K, (b + K) % NBUF).start()

    for jf in range(K):
        gather_copy(jf, jf % NBUF).start()

    @pl.loop(0, n_chunks // NBUF)
    def _grp(o):
        for u in range(NBUF):
            step(o * NBUF + u, u, u % TB)

    for u in range(TB):
        out_copy(0, u).wait()


def kernel(url_ids, table):
    batch, seq = url_ids.shape
    idx = jnp.swapaxes(url_ids, 0, 1).reshape(-1).astype(jnp.int32)
    n = idx.shape[0]
    n_rows = n // NW

    mesh = plsc.VectorSubcoreMesh(
        core_axis_name="c", subcore_axis_name="s", num_cores=NC, num_subcores=NS
    )
    f = pl.kernel(
        _tile_body,
        out_type=jax.ShapeDtypeStruct((seq * D, batch), jnp.float32),
        mesh=mesh,
        compiler_params=pltpu.CompilerParams(
            needs_layout_passes=False, use_tc_tiling_on_sc=False
        ),
        scratch_types=[
            pltpu.VMEM((n_rows + L,), jnp.int32),
            pltpu.VMEM((NBUF, C, D), jnp.float32),
            pltpu.VMEM((TB, D, C + 1), jnp.float32),
            pltpu.VMEM((L,), jnp.int32),
        ]
        + [pltpu.SemaphoreType.DMA] * (NBUF + TB),
    )
    out = f(idx, table)
    return out.reshape(seq, D, batch).transpose(2, 0, 1)
